# Initial kernel scaffold; baseline (speedup 1.0000x reference)
import jax
import jax.numpy as jnp
from jax.experimental import pallas as pl


def kernel(x, edge_index, edge_type, comp1, bases1, root1, bias1, comp2, bases2, root2, bias2):
    raise NotImplementedError("write your pallas kernel here")


# speedup vs baseline: 1.8097x; 1.8097x over previous
import functools

import jax
import jax.numpy as jnp
from jax import lax
from jax.experimental import pallas as pl
from jax.experimental.pallas import tpu as pltpu
from jax.experimental.pallas import tpu_sc as plsc

N = 10000
E = 320000
D = 128
R = 16
B = 8
NR = N * R
NRP = NR + 16
EPT = 20480
EP = EPT * 16
NBP = EPT // 128
CH = 2048
ZR = 250


def _sc_layer_kernel(xvp, esrc, edst, et, z2, o2, s_out, c_out,
                     eb1, eb2, seg2d, idx2d, gbuf, zbuf,
                     sacc, sem_a, sem_b):
  c = lax.axis_index("c")
  sid = lax.axis_index("s")
  base = sid * EPT
  lane = lax.iota(jnp.int32, 16)
  gbase = c * 8

  pltpu.sync_copy(z2, zbuf)

  for ch in range(EPT // CH):
    off = base + ch * CH
    pltpu.sync_copy(edst.at[pl.ds(off, CH)], eb1)
    pltpu.sync_copy(et.at[pl.ds(off, CH)], eb2)

    def segb(i, _, ch=ch):
      row = ch * 16 + lax.shift_right_logical(i, 3)
      col = (i & 7) * 16
      seg2d[row, pl.ds(col, 16)] = (
          eb1[pl.ds(i * 16, 16)] * R + eb2[pl.ds(i * 16, 16)])
      return _
    lax.fori_loop(0, 128, segb, 0)

    pltpu.sync_copy(esrc.at[pl.ds(off, CH)], eb1)

    def idxb(i, _, ch=ch):
      row = ch * 16 + lax.shift_right_logical(i, 3)
      col = (i & 7) * 16
      idx2d[row, pl.ds(col, 16)] = eb1[pl.ds(i * 16, 16)] * 16 + gbase
      return _
    lax.fori_loop(0, 128, idxb, 0)

  for k in range(8):
    idx2d[NBP, pl.ds(k * 16, 16)] = k * 16 + lane

  def zero_own():
    def zb(q, _):
      pltpu.sync_copy(zbuf, sacc.at[pl.ds(sid * 10000 + q * ZR, ZR)])
      return _
    lax.fori_loop(0, 10000 // ZR, zb, 0)

  @pl.when(c == 0)
  def _():
    zero_own()
    pltpu.sync_copy(o2, gbuf.at[0])
    plsc.subcore_barrier()

    def cnt(j, _):
      pltpu.sync_copy(gbuf.at[0], sacc.at[seg2d.at[j]], add=True)
      return _
    lax.fori_loop(0, NBP, cnt, 0)
    plsc.subcore_barrier()
    pltpu.sync_copy(sacc.at[pl.ds(sid * 10000, 10000)],
                    c_out.at[pl.ds(sid * 10000, 10000)])

  for p in range(8):
    table = xvp.at[pl.ds(p, NR)]
    zero_own()
    plsc.subcore_barrier()

    def fire(j, u, sem):
      pltpu.async_copy(table.at[idx2d.at[j]], gbuf.at[u], sem)

    def drain_scatter(j, u, sem):
      pltpu.make_async_copy(table.at[idx2d.at[j]], gbuf.at[u], sem).wait()
      pltpu.sync_copy(gbuf.at[u], sacc.at[seg2d.at[j]], add=True)

    fire(0, 0, sem_a)

    def ring(jj, _):
      fire(2 * jj + 1, 1, sem_b)
      drain_scatter(2 * jj, 0, sem_a)
      fire(2 * jj + 2, 0, sem_a)
      drain_scatter(2 * jj + 1, 1, sem_b)
      return _
    lax.fori_loop(0, NBP // 2, ring, 0)
    pltpu.make_async_copy(table.at[idx2d.at[NBP]], gbuf.at[0], sem_a).wait()

    plsc.subcore_barrier()
    pltpu.sync_copy(sacc.at[pl.ds(sid * 10000, 10000)],
                    s_out.at[c * 8 + p, pl.ds(sid * 10000, 10000)])


def _sc_layer(xvp, esrc, edst, et, z2, o2):
  mesh = plsc.VectorSubcoreMesh(core_axis_name="c", subcore_axis_name="s")
  f = pl.kernel(
      _sc_layer_kernel,
      out_type=(
          jax.ShapeDtypeStruct((16, NR, 8), jnp.float32),
          jax.ShapeDtypeStruct((NR, 8), jnp.float32),
      ),
      mesh=mesh,
      compiler_params=pltpu.CompilerParams(use_tc_tiling_on_sc=False),
      scratch_types=[
          pltpu.VMEM((CH,), jnp.int32),
          pltpu.VMEM((CH,), jnp.int32),
          pltpu.VMEM((NBP, 128), jnp.int32),
          pltpu.VMEM((NBP + 1, 128), jnp.int32),
          pltpu.VMEM((2, 128, 8), jnp.float32),
          pltpu.VMEM((ZR, 8), jnp.float32),
          pltpu.VMEM_SHARED((NRP, 8), jnp.float32),
          pltpu.SemaphoreType.DMA,
          pltpu.SemaphoreType.DMA,
      ],
  )
  return f(xvp, esrc, edst, et, z2, o2)


def _wprep_kernel(comp_ref, basesf_ref, o_ref):
  o_ref[...] = jnp.dot(comp_ref[...], basesf_ref[...],
                       preferred_element_type=jnp.float32)


def _wprep(comp, basesf):
  return pl.pallas_call(
      _wprep_kernel,
      out_shape=jax.ShapeDtypeStruct((R, D * D), jnp.float32),
  )(comp, basesf)


BN = 400


def _tc_dense_kernel(relu, s_ref, c_ref, x_ref, w_ref, r_ref, b_ref, o_ref):
  inv = 1.0 / jnp.maximum(c_ref[...], 1.0)
  sn = s_ref[...] * inv
  sn3 = sn.reshape(BN, R, D)
  acc = jnp.dot(x_ref[...], r_ref[...], preferred_element_type=jnp.float32)
  for r in range(R):
    acc += jnp.dot(sn3[:, r, :], w_ref[r * D:(r + 1) * D, :],
                   preferred_element_type=jnp.float32)
  out = acc + b_ref[...]
  if relu:
    out = jnp.maximum(out, 0.0)
  o_ref[...] = out


def _tc_dense(s, c2, x, wf, root, bias2, relu):
  grid = (N // BN,)
  return pl.pallas_call(
      functools.partial(_tc_dense_kernel, relu),
      grid=grid,
      in_specs=[
          pl.BlockSpec((BN * R, D), lambda i: (i, 0)),
          pl.BlockSpec((BN * R, 1), lambda i: (i, 0)),
          pl.BlockSpec((BN, D), lambda i: (i, 0)),
          pl.BlockSpec((R * D, D), lambda i: (0, 0)),
          pl.BlockSpec((D, D), lambda i: (0, 0)),
          pl.BlockSpec((1, D), lambda i: (0, 0)),
      ],
      out_specs=pl.BlockSpec((BN, D), lambda i: (i, 0)),
      out_shape=jax.ShapeDtypeStruct((N, D), jnp.float32),
  )(s, c2, x, wf, root, bias2)


def kernel(x, edge_index, edge_type, comp1, bases1, root1, bias1,
           comp2, bases2, root2, bias2):
  z2 = jnp.zeros((ZR, 8), jnp.float32)
  o2 = jnp.ones((128, 8), jnp.float32)

  wf1 = _wprep(comp1, bases1.reshape(B, D * D)).reshape(R * D, D)
  wf2 = _wprep(comp2, bases2.reshape(B, D * D)).reshape(R * D, D)
  b1 = bias1.reshape(1, D)
  b2 = bias2.reshape(1, D)

  npad = EP - E
  pad_i = jnp.arange(npad, dtype=jnp.int32)
  esrc = jnp.concatenate([edge_index[0], pad_i % N])
  edst = jnp.concatenate([edge_index[1], jnp.full((npad,), N, jnp.int32)])
  etp = jnp.concatenate([edge_type, pad_i % R])
  pad8 = jnp.zeros((8, 8), jnp.float32)

  xvp = jnp.concatenate([x.reshape(NR, 8), pad8])
  s1, c1 = _sc_layer(xvp, esrc, edst, etp, z2, o2)
  s1 = s1.transpose(1, 0, 2).reshape(NR, D)
  h = _tc_dense(s1, c1[:, :1], x, wf1, root1, b1, relu=True)

  hvp = jnp.concatenate([h.reshape(NR, 8), pad8])
  s2, c2 = _sc_layer(hvp, esrc, edst, etp, z2, o2)
  s2 = s2.transpose(1, 0, 2).reshape(NR, D)
  out = _tc_dense(s2, c2[:, :1], h, wf2, root2, b2, relu=False)
  return out
